# Initial kernel scaffold; baseline (speedup 1.0000x reference)
import jax
import jax.numpy as jnp
from jax.experimental import pallas as pl


def kernel(x, edge_index, W_l, b_l, W_r):
    raise NotImplementedError("write your pallas kernel here")


# speedup vs baseline: 4.2784x; 4.2784x over previous
import functools

import jax
import jax.numpy as jnp
from jax import lax
from jax.experimental import pallas as pl
from jax.experimental.pallas import tpu as pltpu
from jax.experimental.pallas import tpu_sc as plsc

N_NODES = 10000
D = 128
E = 320000

NC = 2
NS = 16
NW = NC * NS

CHUNK = 128
N_ITERS = 79
EDGES_PER_W = CHUNK * N_ITERS
E_PAD = NW * EDGES_PER_W
N_ACC = 10240
ROWS_PER_TILE = N_ACC // NS
PAD_DST = N_NODES


def _sc_aggregate(x, row, col, zblock):
    mesh = plsc.VectorSubcoreMesh(core_axis_name="c", subcore_axis_name="s")

    @functools.partial(
        pl.kernel,
        mesh=mesh,
        out_type=jax.ShapeDtypeStruct((NC, N_ACC, D), jnp.float32),
        scratch_types=[
            pltpu.VMEM((CHUNK,), jnp.int32),
            pltpu.VMEM((CHUNK,), jnp.int32),
            pltpu.VMEM((CHUNK, D), jnp.float32),
            pltpu.VMEM_SHARED((N_ACC, D), jnp.float32),
            pltpu.SemaphoreType.DMA,
        ],
    )
    def body(x_hbm, row_hbm, col_hbm, z_hbm, out_hbm, idx_r, idx_c, rows_v,
             acc_sh, sem):
        cid = lax.axis_index("c")
        sid = lax.axis_index("s")
        wid = cid * NS + sid

        pltpu.sync_copy(z_hbm, rows_v)
        r0 = sid * ROWS_PER_TILE
        for b in range(ROWS_PER_TILE // CHUNK):
            pltpu.sync_copy(rows_v, acc_sh.at[pl.ds(r0 + b * CHUNK, CHUNK)])
        plsc.subcore_barrier()

        def step(i, carry):
            base = wid * EDGES_PER_W + i * CHUNK
            pltpu.sync_copy(row_hbm.at[pl.ds(base, CHUNK)], idx_r)
            pltpu.sync_copy(col_hbm.at[pl.ds(base, CHUNK)], idx_c)
            pltpu.async_copy(x_hbm.at[idx_r], rows_v, sem).wait()
            pltpu.sync_copy(rows_v, acc_sh.at[idx_c], add=True)
            return carry

        lax.fori_loop(0, N_ITERS, step, 0)
        plsc.subcore_barrier()

        pltpu.sync_copy(acc_sh.at[pl.ds(r0, ROWS_PER_TILE)],
                        out_hbm.at[cid, pl.ds(r0, ROWS_PER_TILE)])

    return body(x, row, col, zblock)


def _dense_body(p0_ref, p1_ref, x_ref, wl_ref, wr_ref, b_ref, o_ref):
    agg = p0_ref[...] + p1_ref[...]
    o_ref[...] = (
        lax.dot_general(agg, wl_ref[...], (((1,), (1,)), ((), ())),
                        preferred_element_type=jnp.float32)
        + lax.dot_general(x_ref[...], wr_ref[...], (((1,), (1,)), ((), ())),
                          preferred_element_type=jnp.float32)
        + b_ref[...]
    )


def kernel(x, edge_index, W_l, b_l, W_r):
    row = edge_index[0]
    col = edge_index[1]
    npad = E_PAD - E
    row = jnp.concatenate([row, jnp.zeros((npad,), jnp.int32)])
    col = jnp.concatenate([col, jnp.full((npad,), PAD_DST, jnp.int32)])
    zblock = jnp.zeros((CHUNK, D), jnp.float32)

    p = _sc_aggregate(x, row, col, zblock)

    blk = 1000
    grid = (N_NODES // blk,)
    out = pl.pallas_call(
        _dense_body,
        grid=grid,
        in_specs=[
            pl.BlockSpec((blk, D), lambda i: (i, 0)),
            pl.BlockSpec((blk, D), lambda i: (i, 0)),
            pl.BlockSpec((blk, D), lambda i: (i, 0)),
            pl.BlockSpec((D, D), lambda i: (0, 0)),
            pl.BlockSpec((D, D), lambda i: (0, 0)),
            pl.BlockSpec((1, D), lambda i: (0, 0)),
        ],
        out_specs=pl.BlockSpec((blk, D), lambda i: (i, 0)),
        out_shape=jax.ShapeDtypeStruct((N_NODES, D), jnp.float32),
    )(p[0], p[1], x, W_l, W_r, b_l.reshape(1, D))
    return out
